# Initial kernel scaffold; baseline (speedup 1.0000x reference)
import jax
import jax.numpy as jnp
from jax.experimental import pallas as pl


def kernel(node_feats, edge_feats, edge_index, Wsg, bsg, Wdg, bdg, Weg, beg, Wsu, bsu, Wdu, bdu, gamma_n, beta_n, gamma_e, beta_e):
    raise NotImplementedError("write your pallas kernel here")


# speedup vs baseline: 1.1060x; 1.1060x over previous
import functools

import jax
import jax.numpy as jnp
from jax import lax
from jax.experimental import pallas as pl
from jax.experimental.pallas import tpu as pltpu
from jax.experimental.pallas import tpu_sc as plsc

N = 10000
E = 320000
D = 128
NC = 2
NS = 16
L = 16
NW = NC * NS
EPW = E // NW
CB = 80
NCHUNK = EPW // CB
N_PAD = 10240
RPT = N_PAD // NS
ZR = 64
SCB_ROWS = max(CB, ZR)

_NBLK = 1000
_EBLK = 512


def _node_mm_body(x_ref, wsg_ref, bsg_ref, wdg_ref, bdg_ref, wdu_ref, bdu_ref,
                  wsu_ref, bsu_ref, tsrc_ref, tdst_ref, ah_ref):
    x = x_ref[...]
    rsg = jnp.dot(x, wsg_ref[...], preferred_element_type=jnp.float32) + bsg_ref[...]
    rdg = jnp.dot(x, wdg_ref[...], preferred_element_type=jnp.float32) + bdg_ref[...]
    rdu = jnp.dot(x, wdu_ref[...], preferred_element_type=jnp.float32) + bdu_ref[...]
    rsu = jnp.dot(x, wsu_ref[...], preferred_element_type=jnp.float32) + bsu_ref[...]
    tsrc_ref[:, :D] = rsg
    tsrc_ref[:, D:] = rdu
    tdst_ref[...] = rdg
    ah_ref[...] = rsu


def _node_matmuls(node_feats, Wsg, bsg, Wdg, bdg, Wdu, bdu, Wsu, bsu):
    nblk = N // _NBLK
    full_w = pl.BlockSpec((D, D), lambda i: (0, 0))
    full_b = pl.BlockSpec((1, D), lambda i: (0, 0))
    return pl.pallas_call(
        _node_mm_body,
        grid=(nblk,),
        in_specs=[
            pl.BlockSpec((_NBLK, D), lambda i: (i, 0)),
            full_w, full_b, full_w, full_b, full_w, full_b, full_w, full_b,
        ],
        out_specs=[pl.BlockSpec((_NBLK, 2 * D), lambda i: (i, 0)),
                   pl.BlockSpec((_NBLK, D), lambda i: (i, 0)),
                   pl.BlockSpec((_NBLK, D), lambda i: (i, 0))],
        out_shape=[
            jax.ShapeDtypeStruct((N, 2 * D), jnp.float32),
            jax.ShapeDtypeStruct((N, D), jnp.float32),
            jax.ShapeDtypeStruct((N, D), jnp.float32),
        ],
    )(node_feats, Wsg, bsg.reshape(1, D), Wdg, bdg.reshape(1, D),
      Wdu, bdu.reshape(1, D), Wsu, bsu.reshape(1, D))


def _edge_mm_body(x_ref, w_ref, b_ref, o_ref):
    o_ref[...] = (jnp.dot(x_ref[...], w_ref[...],
                          preferred_element_type=jnp.float32) + b_ref[...])


def _edge_gate(edge_feats, Weg, beg):
    eblk = E // _EBLK
    return pl.pallas_call(
        _edge_mm_body,
        grid=(eblk,),
        in_specs=[
            pl.BlockSpec((_EBLK, D), lambda i: (i, 0)),
            pl.BlockSpec((D, D), lambda i: (0, 0)),
            pl.BlockSpec((1, D), lambda i: (0, 0)),
        ],
        out_specs=pl.BlockSpec((_EBLK, D), lambda i: (i, 0)),
        out_shape=jax.ShapeDtypeStruct((E, D), jnp.float32),
    )(edge_feats, Weg, beg.reshape(1, D))


def _sc_edge_body(src_hbm, dst_hbm, tsrc_hbm, tdst_hbm, eg_hbm,
                  m_hbm, accm_hbm, accs_hbm, stats_hbm,
                  shared_acc, idx_src, idx_dst,
                  gsrc, egb, scb, stats_v,
                  sem1, sem2, sem3):
    c = lax.axis_index("c")
    s = lax.axis_index("s")
    base = (c * NS + s) * EPW
    zero16 = jnp.zeros((L,), jnp.float32)

    for j in range(2 * D // L):
        stats_v[0, pl.ds(j * L, L)] = zero16

    def _zrow(i, carry):
        for j in range(D // L):
            scb[i, pl.ds(j * L, L)] = zero16
        return carry
    lax.fori_loop(0, SCB_ROWS, _zrow, 0)

    def _zero_shared():
        for t in range(RPT // ZR):
            pltpu.sync_copy(scb.at[pl.ds(0, ZR)],
                            shared_acc.at[pl.ds(s * RPT + t * ZR, ZR)])

    _zero_shared()
    plsc.subcore_barrier()

    def _chunk1(k, carry):
        e0 = base + k * CB
        pltpu.sync_copy(src_hbm.at[pl.ds(e0, CB)], idx_src)
        pltpu.sync_copy(dst_hbm.at[pl.ds(e0, CB)], idx_dst)
        d1 = pltpu.async_copy(tsrc_hbm.at[idx_src], gsrc, sem1)
        d2 = pltpu.async_copy(tdst_hbm.at[idx_dst], scb, sem2)
        d3 = pltpu.async_copy(eg_hbm.at[pl.ds(e0, CB)], egb, sem3)
        d1.wait()
        d2.wait()
        d3.wait()

        def _row(i, rc):
            for j in range(D // L):
                sl = pl.ds(j * L, L)
                sh = pl.ds(D + j * L, L)
                m = gsrc[i, sl] + scb[i, sl] + egb[i, sl]
                egb[i, sl] = m
                sg = 1.0 / (1.0 + jnp.exp(-m))
                scb[i, sl] = gsrc[i, sh] * sg
                stats_v[0, sl] = stats_v[0, sl] + m
                stats_v[0, sh] = stats_v[0, sh] + m * m
            return rc
        lax.fori_loop(0, CB, _row, 0)

        pltpu.sync_copy(egb, m_hbm.at[pl.ds(e0, CB)])
        pltpu.sync_copy(scb, shared_acc.at[idx_dst], add=True)
        return carry

    lax.fori_loop(0, NCHUNK, _chunk1, 0)
    plsc.subcore_barrier()
    pltpu.sync_copy(shared_acc.at[pl.ds(s * RPT, RPT)],
                    accm_hbm.at[c, pl.ds(s * RPT, RPT)])
    pltpu.sync_copy(stats_v, stats_hbm.at[c, s])
    plsc.subcore_barrier()

    lax.fori_loop(0, SCB_ROWS, _zrow, 0)
    _zero_shared()
    plsc.subcore_barrier()

    def _chunk2(k, carry):
        e0 = base + k * CB
        pltpu.sync_copy(dst_hbm.at[pl.ds(e0, CB)], idx_dst)
        d3 = pltpu.async_copy(m_hbm.at[pl.ds(e0, CB)], egb, sem3)
        d3.wait()

        def _row(i, rc):
            for j in range(D // L):
                sl = pl.ds(j * L, L)
                scb[i, sl] = 1.0 / (1.0 + jnp.exp(-egb[i, sl]))
            return rc
        lax.fori_loop(0, CB, _row, 0)

        pltpu.sync_copy(scb, shared_acc.at[idx_dst], add=True)
        return carry

    lax.fori_loop(0, NCHUNK, _chunk2, 0)
    plsc.subcore_barrier()
    pltpu.sync_copy(shared_acc.at[pl.ds(s * RPT, RPT)],
                    accs_hbm.at[c, pl.ds(s * RPT, RPT)])


def _sc_edge(src, dst, tsrc, tdst, eg):
    mesh = plsc.VectorSubcoreMesh(core_axis_name="c", subcore_axis_name="s",
                                  num_cores=NC, num_subcores=NS)
    fn = pl.kernel(
        _sc_edge_body,
        out_type=[
            jax.ShapeDtypeStruct((E, D), jnp.float32),
            jax.ShapeDtypeStruct((NC, N_PAD, D), jnp.float32),
            jax.ShapeDtypeStruct((NC, N_PAD, D), jnp.float32),
            jax.ShapeDtypeStruct((NC, NS, 1, 2 * D), jnp.float32),
        ],
        mesh=mesh,
        scratch_types=[
            pltpu.VMEM_SHARED((N_PAD, D), jnp.float32),
            pltpu.VMEM((CB,), jnp.int32),
            pltpu.VMEM((CB,), jnp.int32),
            pltpu.VMEM((CB, 2 * D), jnp.float32),
            pltpu.VMEM((CB, D), jnp.float32),
            pltpu.VMEM((SCB_ROWS, D), jnp.float32),
            pltpu.VMEM((1, 2 * D), jnp.float32),
            pltpu.SemaphoreType.DMA,
            pltpu.SemaphoreType.DMA,
            pltpu.SemaphoreType.DMA,
        ],
    )
    return fn(src, dst, tsrc, tdst, eg)


def _node_final_body(ah_ref, accm_ref, accs_ref, nf_ref, g_ref, b_ref, x_ref):
    accm = accm_ref[...]
    accs = accs_ref[...]
    num = accm[0, :N, :] + accm[1, :N, :]
    den = accs[0, :N, :] + accs[1, :N, :]
    xp = ah_ref[...] + num / (den + 1e-6)
    mu = jnp.mean(xp, axis=0, keepdims=True)
    var = jnp.mean(xp * xp, axis=0, keepdims=True) - mu * mu
    xn = (xp - mu) * lax.rsqrt(var + 1e-5) * g_ref[...] + b_ref[...]
    x_ref[...] = nf_ref[...] + xn * jax.nn.sigmoid(xn)


def _node_final(ah, accm, accs, node_feats, gamma_n, beta_n):
    return pl.pallas_call(
        _node_final_body,
        out_shape=jax.ShapeDtypeStruct((N, D), jnp.float32),
    )(ah, accm, accs, node_feats, gamma_n.reshape(1, D), beta_n.reshape(1, D))


def _edge_final_body(m_ref, ef_ref, stats_ref, g_ref, b_ref, y_ref):
    st = stats_ref[...].reshape(NW, 2 * D)
    mean = jnp.sum(st[:, :D], axis=0, keepdims=True) / E
    msq = jnp.sum(st[:, D:], axis=0, keepdims=True) / E
    var = msq - mean * mean
    m = m_ref[...]
    yn = (m - mean) * lax.rsqrt(var + 1e-5) * g_ref[...] + b_ref[...]
    y_ref[...] = ef_ref[...] + yn * jax.nn.sigmoid(yn)


def _edge_final(m, edge_feats, stats, gamma_e, beta_e):
    eblk = E // _EBLK
    return pl.pallas_call(
        _edge_final_body,
        grid=(eblk,),
        in_specs=[
            pl.BlockSpec((_EBLK, D), lambda i: (i, 0)),
            pl.BlockSpec((_EBLK, D), lambda i: (i, 0)),
            pl.BlockSpec((NC, NS, 1, 2 * D), lambda i: (0, 0, 0, 0)),
            pl.BlockSpec((1, D), lambda i: (0, 0)),
            pl.BlockSpec((1, D), lambda i: (0, 0)),
        ],
        out_specs=pl.BlockSpec((_EBLK, D), lambda i: (i, 0)),
        out_shape=jax.ShapeDtypeStruct((E, D), jnp.float32),
    )(m, edge_feats, stats, gamma_e.reshape(1, D), beta_e.reshape(1, D))


def kernel(node_feats, edge_feats, edge_index, Wsg, bsg, Wdg, bdg, Weg, beg,
           Wsu, bsu, Wdu, bdu, gamma_n, beta_n, gamma_e, beta_e):
    src = edge_index[0].astype(jnp.int32)
    dst = edge_index[1].astype(jnp.int32)

    tsrc, tdst, ah = _node_matmuls(
        node_feats, Wsg, bsg, Wdg, bdg, Wdu, bdu, Wsu, bsu)
    eg = _edge_gate(edge_feats, Weg, beg)

    m, accm, accs, stats = _sc_edge(src, dst, tsrc, tdst, eg)

    x = _node_final(ah, accm, accs, node_feats, gamma_n, beta_n)
    y = _edge_final(m, edge_feats, stats, gamma_e, beta_e)
    return (x, y)


# speedup vs baseline: 1.1909x; 1.0768x over previous
import functools

import jax
import jax.numpy as jnp
from jax import lax
from jax.experimental import pallas as pl
from jax.experimental.pallas import tpu as pltpu
from jax.experimental.pallas import tpu_sc as plsc

N = 10000
E = 320000
D = 128
NC = 2
NS = 16
L = 16
NW = NC * NS
EPW = E // NW
CB = 40
NCHUNK = EPW // CB
N_PAD = 10240
RPT = N_PAD // NS
IBLK = 25
CPB = NCHUNK // IBLK

_NBLK = 1000
_EBLK = 512


def _node_mm_body(x_ref, wsg_ref, bsg_ref, wdg_ref, bdg_ref, wdu_ref, bdu_ref,
                  wsu_ref, bsu_ref, tsrc_ref, tdst_ref, ah_ref):
    x = x_ref[...]
    rsg = jnp.dot(x, wsg_ref[...], preferred_element_type=jnp.float32) + bsg_ref[...]
    rdg = jnp.dot(x, wdg_ref[...], preferred_element_type=jnp.float32) + bdg_ref[...]
    rdu = jnp.dot(x, wdu_ref[...], preferred_element_type=jnp.float32) + bdu_ref[...]
    rsu = jnp.dot(x, wsu_ref[...], preferred_element_type=jnp.float32) + bsu_ref[...]
    tsrc_ref[:, :D] = rsg
    tsrc_ref[:, D:] = rdu
    tdst_ref[...] = rdg
    ah_ref[...] = rsu


def _node_matmuls(node_feats, Wsg, bsg, Wdg, bdg, Wdu, bdu, Wsu, bsu):
    nblk = N // _NBLK
    full_w = pl.BlockSpec((D, D), lambda i: (0, 0))
    full_b = pl.BlockSpec((1, D), lambda i: (0, 0))
    return pl.pallas_call(
        _node_mm_body,
        grid=(nblk,),
        in_specs=[
            pl.BlockSpec((_NBLK, D), lambda i: (i, 0)),
            full_w, full_b, full_w, full_b, full_w, full_b, full_w, full_b,
        ],
        out_specs=[pl.BlockSpec((_NBLK, 2 * D), lambda i: (i, 0)),
                   pl.BlockSpec((_NBLK, D), lambda i: (i, 0)),
                   pl.BlockSpec((_NBLK, D), lambda i: (i, 0))],
        out_shape=[
            jax.ShapeDtypeStruct((N, 2 * D), jnp.float32),
            jax.ShapeDtypeStruct((N, D), jnp.float32),
            jax.ShapeDtypeStruct((N, D), jnp.float32),
        ],
    )(node_feats, Wsg, bsg.reshape(1, D), Wdg, bdg.reshape(1, D),
      Wdu, bdu.reshape(1, D), Wsu, bsu.reshape(1, D))


def _edge_mm_body(x_ref, w_ref, b_ref, o_ref):
    o_ref[...] = (jnp.dot(x_ref[...], w_ref[...],
                          preferred_element_type=jnp.float32) + b_ref[...])


def _edge_gate(edge_feats, Weg, beg):
    eblk = E // _EBLK
    return pl.pallas_call(
        _edge_mm_body,
        grid=(eblk,),
        in_specs=[
            pl.BlockSpec((_EBLK, D), lambda i: (i, 0)),
            pl.BlockSpec((D, D), lambda i: (0, 0)),
            pl.BlockSpec((1, D), lambda i: (0, 0)),
        ],
        out_specs=pl.BlockSpec((_EBLK, D), lambda i: (i, 0)),
        out_shape=jax.ShapeDtypeStruct((E, D), jnp.float32),
    )(edge_feats, Weg, beg.reshape(1, D))


def _sc_edge_body(src_hbm, dst_hbm, tsrc_hbm, tdst_hbm, eg_hbm,
                  m_hbm, accm_hbm, accs_hbm, stats_hbm,
                  shared_acc, srcv, dstv,
                  gsrc2, egb2, scb2, stats_v,
                  sga0, sgb0, sge0, sga1, sgb1, sge1,
                  swm0, sws0, swm1, sws1):
    c = lax.axis_index("c")
    s = lax.axis_index("s")
    w = c * NS + s
    base = w * EPW
    zero16 = jnp.zeros((L,), jnp.float32)
    semg = ((sga0, sgb0, sge0), (sga1, sgb1, sge1))
    semw = ((swm0, sws0), (swm1, sws1))
    gsrc = (gsrc2.at[0], gsrc2.at[1])
    egb = (egb2.at[0], egb2.at[1])
    scb = (scb2.at[0], scb2.at[1])

    for r in range(2):
        for j in range(D // L):
            stats_v[r, pl.ds(j * L, L)] = zero16

    def _zrow(i, carry):
        for b in range(2):
            for j in range(D // L):
                scb2[b, i, pl.ds(j * L, L)] = zero16
        return carry

    def _zero_shared():
        lax.fori_loop(0, CB, _zrow, 0)
        for t in range(RPT // CB):
            pltpu.sync_copy(scb2.at[0],
                            shared_acc.at[pl.ds(s * RPT + t * CB, CB)])

    def _pipeline(fire, compute, fire_w):
        def _pair(p, carry):
            kk = 2 * p
            da = fire(kk, 0)
            for d in da:
                d.wait()
            db = fire(kk + 1, 1)
            compute(0)
            wa = fire_w(kk, 0)
            for d in db:
                d.wait()
            compute(1)
            for d in wa:
                d.wait()
            wb = fire_w(kk + 1, 1)
            for d in wb:
                d.wait()
            return carry
        lax.fori_loop(0, CPB // 2, _pair, 0)

    def _phase1_block(blk):
        k0 = blk * CPB
        pltpu.sync_copy(src_hbm.at[w, blk], srcv)
        pltpu.sync_copy(dst_hbm.at[w, blk], dstv)

        def fire(kk, b):
            return [
                pltpu.async_copy(tsrc_hbm.at[srcv.at[kk]], gsrc[b],
                                 semg[b][0]),
                pltpu.async_copy(tdst_hbm.at[dstv.at[kk]], scb[b],
                                 semg[b][1]),
                pltpu.async_copy(eg_hbm.at[pl.ds(base + (k0 + kk) * CB, CB)],
                                 egb[b], semg[b][2]),
            ]

        def compute(b):
            def _row(i, rc):
                for j in range(D // L):
                    sl = pl.ds(j * L, L)
                    sh = pl.ds(D + j * L, L)
                    m = gsrc[b][i, sl] + scb[b][i, sl] + egb[b][i, sl]
                    egb[b][i, sl] = m
                    sg = 1.0 / (1.0 + jnp.exp(-m))
                    scb[b][i, sl] = gsrc[b][i, sh] * sg
                    stats_v[0, sl] = stats_v[0, sl] + m
                    stats_v[1, sl] = stats_v[1, sl] + m * m
                return rc
            lax.fori_loop(0, CB, _row, 0)

        def fire_w(kk, b):
            return [
                pltpu.async_copy(egb[b],
                                 m_hbm.at[pl.ds(base + (k0 + kk) * CB, CB)],
                                 semw[b][0]),
                pltpu.async_copy(scb[b], shared_acc.at[dstv.at[kk]],
                                 semw[b][1], add=True),
            ]

        _pipeline(fire, compute, fire_w)

    def _phase2_block(blk):
        k0 = blk * CPB
        pltpu.sync_copy(dst_hbm.at[w, blk], dstv)

        def fire(kk, b):
            return [
                pltpu.async_copy(m_hbm.at[pl.ds(base + (k0 + kk) * CB, CB)],
                                 egb[b], semg[b][2]),
            ]

        def compute(b):
            def _row(i, rc):
                for j in range(D // L):
                    sl = pl.ds(j * L, L)
                    scb[b][i, sl] = 1.0 / (1.0 + jnp.exp(-egb[b][i, sl]))
                return rc
            lax.fori_loop(0, CB, _row, 0)

        def fire_w(kk, b):
            return [
                pltpu.async_copy(scb[b], shared_acc.at[dstv.at[kk]],
                                 semw[b][1], add=True),
            ]

        _pipeline(fire, compute, fire_w)

    _zero_shared()
    plsc.subcore_barrier()

    def _p1(blk, carry):
        _phase1_block(blk)
        return carry
    lax.fori_loop(0, IBLK, _p1, 0)
    plsc.subcore_barrier()
    pltpu.sync_copy(shared_acc.at[pl.ds(s * RPT, RPT)],
                    accm_hbm.at[c, pl.ds(s * RPT, RPT)])
    pltpu.sync_copy(stats_v, stats_hbm.at[c, s])
    _zero_shared()
    plsc.subcore_barrier()

    def _p2(blk, carry):
        _phase2_block(blk)
        return carry
    lax.fori_loop(0, IBLK, _p2, 0)
    plsc.subcore_barrier()
    pltpu.sync_copy(shared_acc.at[pl.ds(s * RPT, RPT)],
                    accs_hbm.at[c, pl.ds(s * RPT, RPT)])


def _sc_edge(src4, dst4, tsrc, tdst, eg):
    mesh = plsc.VectorSubcoreMesh(core_axis_name="c", subcore_axis_name="s",
                                  num_cores=NC, num_subcores=NS)
    fn = pl.kernel(
        _sc_edge_body,
        out_type=[
            jax.ShapeDtypeStruct((E, D), jnp.float32),
            jax.ShapeDtypeStruct((NC, N_PAD, D), jnp.float32),
            jax.ShapeDtypeStruct((NC, N_PAD, D), jnp.float32),
            jax.ShapeDtypeStruct((NC, NS, 2, D), jnp.float32),
        ],
        mesh=mesh,
        scratch_types=[
            pltpu.VMEM_SHARED((N_PAD, D), jnp.float32),
            pltpu.VMEM((CPB, CB), jnp.int32),
            pltpu.VMEM((CPB, CB), jnp.int32),
            pltpu.VMEM((2, CB, 2 * D), jnp.float32),
            pltpu.VMEM((2, CB, D), jnp.float32),
            pltpu.VMEM((2, CB, D), jnp.float32),
            pltpu.VMEM((2, D), jnp.float32),
            pltpu.SemaphoreType.DMA,
            pltpu.SemaphoreType.DMA,
            pltpu.SemaphoreType.DMA,
            pltpu.SemaphoreType.DMA,
            pltpu.SemaphoreType.DMA,
            pltpu.SemaphoreType.DMA,
            pltpu.SemaphoreType.DMA,
            pltpu.SemaphoreType.DMA,
            pltpu.SemaphoreType.DMA,
            pltpu.SemaphoreType.DMA,
        ],
    )
    return fn(src4, dst4, tsrc, tdst, eg)


def _node_final_body(ah_ref, accm_ref, accs_ref, nf_ref, g_ref, b_ref, x_ref):
    accm = accm_ref[...]
    accs = accs_ref[...]
    num = accm[0, :N, :] + accm[1, :N, :]
    den = accs[0, :N, :] + accs[1, :N, :]
    xp = ah_ref[...] + num / (den + 1e-6)
    mu = jnp.mean(xp, axis=0, keepdims=True)
    var = jnp.mean(xp * xp, axis=0, keepdims=True) - mu * mu
    xn = (xp - mu) * lax.rsqrt(var + 1e-5) * g_ref[...] + b_ref[...]
    x_ref[...] = nf_ref[...] + xn * jax.nn.sigmoid(xn)


def _node_final(ah, accm, accs, node_feats, gamma_n, beta_n):
    return pl.pallas_call(
        _node_final_body,
        out_shape=jax.ShapeDtypeStruct((N, D), jnp.float32),
    )(ah, accm, accs, node_feats, gamma_n.reshape(1, D), beta_n.reshape(1, D))


def _edge_final_body(m_ref, ef_ref, stats_ref, g_ref, b_ref, y_ref):
    st = stats_ref[...].reshape(NW, 2, D)
    mean = jnp.sum(st[:, 0, :], axis=0, keepdims=True) / E
    msq = jnp.sum(st[:, 1, :], axis=0, keepdims=True) / E
    var = msq - mean * mean
    m = m_ref[...]
    yn = (m - mean) * lax.rsqrt(var + 1e-5) * g_ref[...] + b_ref[...]
    y_ref[...] = ef_ref[...] + yn * jax.nn.sigmoid(yn)


def _edge_final(m, edge_feats, stats, gamma_e, beta_e):
    eblk = E // _EBLK
    return pl.pallas_call(
        _edge_final_body,
        grid=(eblk,),
        in_specs=[
            pl.BlockSpec((_EBLK, D), lambda i: (i, 0)),
            pl.BlockSpec((_EBLK, D), lambda i: (i, 0)),
            pl.BlockSpec((NC, NS, 2, D), lambda i: (0, 0, 0, 0)),
            pl.BlockSpec((1, D), lambda i: (0, 0)),
            pl.BlockSpec((1, D), lambda i: (0, 0)),
        ],
        out_specs=pl.BlockSpec((_EBLK, D), lambda i: (i, 0)),
        out_shape=jax.ShapeDtypeStruct((E, D), jnp.float32),
    )(m, edge_feats, stats, gamma_e.reshape(1, D), beta_e.reshape(1, D))


def kernel(node_feats, edge_feats, edge_index, Wsg, bsg, Wdg, bdg, Weg, beg,
           Wsu, bsu, Wdu, bdu, gamma_n, beta_n, gamma_e, beta_e):
    src = edge_index[0].astype(jnp.int32)
    dst = edge_index[1].astype(jnp.int32)

    tsrc, tdst, ah = _node_matmuls(
        node_feats, Wsg, bsg, Wdg, bdg, Wdu, bdu, Wsu, bsu)
    eg = _edge_gate(edge_feats, Weg, beg)

    m, accm, accs, stats = _sc_edge(src.reshape(NW, IBLK, CPB, CB),
                                    dst.reshape(NW, IBLK, CPB, CB),
                                    tsrc, tdst, eg)

    x = _node_final(ah, accm, accs, node_feats, gamma_n, beta_n)
    y = _edge_final(m, edge_feats, stats, gamma_e, beta_e)
    return (x, y)


# speedup vs baseline: 1.3564x; 1.1389x over previous
import functools

import jax
import jax.numpy as jnp
from jax import lax
from jax.experimental import pallas as pl
from jax.experimental.pallas import tpu as pltpu
from jax.experimental.pallas import tpu_sc as plsc

N = 10000
E = 320000
D = 128
NC = 2
NS = 16
L = 16
NW = NC * NS
EPW = E // NW
CB = 40
NCHUNK = EPW // CB
N_PAD = 10240
RPT = N_PAD // NS
IBLK = 25
CPB = NCHUNK // IBLK

_NBLK = 1000
_EBLK = 512


def _node_mm_body(x_ref, wsg_ref, bsg_ref, wdg_ref, bdg_ref, wdu_ref, bdu_ref,
                  wsu_ref, bsu_ref, tsrc_ref, tdst_ref, ah_ref):
    x = x_ref[...]
    rsg = jnp.dot(x, wsg_ref[...], preferred_element_type=jnp.float32) + bsg_ref[...]
    rdg = jnp.dot(x, wdg_ref[...], preferred_element_type=jnp.float32) + bdg_ref[...]
    rdu = jnp.dot(x, wdu_ref[...], preferred_element_type=jnp.float32) + bdu_ref[...]
    rsu = jnp.dot(x, wsu_ref[...], preferred_element_type=jnp.float32) + bsu_ref[...]
    tsrc_ref[:, :D] = rsg
    tsrc_ref[:, D:] = rdu
    tdst_ref[...] = rdg
    ah_ref[...] = rsu


def _node_matmuls(node_feats, Wsg, bsg, Wdg, bdg, Wdu, bdu, Wsu, bsu):
    nblk = N // _NBLK
    full_w = pl.BlockSpec((D, D), lambda i: (0, 0))
    full_b = pl.BlockSpec((1, D), lambda i: (0, 0))
    return pl.pallas_call(
        _node_mm_body,
        grid=(nblk,),
        in_specs=[
            pl.BlockSpec((_NBLK, D), lambda i: (i, 0)),
            full_w, full_b, full_w, full_b, full_w, full_b, full_w, full_b,
        ],
        out_specs=[pl.BlockSpec((_NBLK, 2 * D), lambda i: (i, 0)),
                   pl.BlockSpec((_NBLK, D), lambda i: (i, 0)),
                   pl.BlockSpec((_NBLK, D), lambda i: (i, 0))],
        out_shape=[
            jax.ShapeDtypeStruct((N, 2 * D), jnp.float32),
            jax.ShapeDtypeStruct((N, D), jnp.float32),
            jax.ShapeDtypeStruct((N, D), jnp.float32),
        ],
    )(node_feats, Wsg, bsg.reshape(1, D), Wdg, bdg.reshape(1, D),
      Wdu, bdu.reshape(1, D), Wsu, bsu.reshape(1, D))


def _edge_mm_body(x_ref, w_ref, b_ref, o_ref):
    o_ref[...] = (jnp.dot(x_ref[...], w_ref[...],
                          preferred_element_type=jnp.float32) + b_ref[...])


def _edge_gate(edge_feats, Weg, beg):
    eblk = E // _EBLK
    return pl.pallas_call(
        _edge_mm_body,
        grid=(eblk,),
        in_specs=[
            pl.BlockSpec((_EBLK, D), lambda i: (i, 0)),
            pl.BlockSpec((D, D), lambda i: (0, 0)),
            pl.BlockSpec((1, D), lambda i: (0, 0)),
        ],
        out_specs=pl.BlockSpec((_EBLK, D), lambda i: (i, 0)),
        out_shape=jax.ShapeDtypeStruct((E, D), jnp.float32),
    )(edge_feats, Weg, beg.reshape(1, D))


def _sc_edge_body(src_hbm, dst_hbm, tsrc_hbm, tdst_hbm, eg_hbm,
                  m_hbm, accm_hbm, accs_hbm, stats_hbm,
                  shared_acc, srcv, dstv,
                  gsrc2, egb2, scb2, stats_v,
                  sga0, sgb0, sge0, sga1, sgb1, sge1,
                  swm0, sws0, swm1, sws1):
    c = lax.axis_index("c")
    s = lax.axis_index("s")
    w = c * NS + s
    base = w * EPW
    zero16 = jnp.zeros((L,), jnp.float32)
    semg = ((sga0, sgb0, sge0), (sga1, sgb1, sge1))
    semw = ((swm0, sws0), (swm1, sws1))
    gsrc = (gsrc2.at[0], gsrc2.at[1])
    egb = (egb2.at[0], egb2.at[1])
    scb = (scb2.at[0], scb2.at[1])

    for r in range(2):
        for j in range(D // L):
            stats_v[r, pl.ds(j * L, L)] = zero16

    def _zrow(i, carry):
        for b in range(2):
            for j in range(D // L):
                scb2[b, i, pl.ds(j * L, L)] = zero16
        return carry

    def _zero_shared():
        lax.fori_loop(0, CB, _zrow, 0)
        for t in range(RPT // CB):
            pltpu.sync_copy(scb2.at[0],
                            shared_acc.at[pl.ds(s * RPT + t * CB, CB)])

    def _pipeline(fire, compute, fire_w):
        def _pair(p, carry):
            kk = 2 * p
            da = fire(kk, 0)
            for d in da:
                d.wait()
            db = fire(kk + 1, 1)
            compute(0)
            wa = fire_w(kk, 0)
            for d in db:
                d.wait()
            compute(1)
            for d in wa:
                d.wait()
            wb = fire_w(kk + 1, 1)
            for d in wb:
                d.wait()
            return carry
        lax.fori_loop(0, CPB // 2, _pair, 0)

    def _phase1_block(blk):
        k0 = blk * CPB
        pltpu.sync_copy(src_hbm.at[w, blk], srcv)
        pltpu.sync_copy(dst_hbm.at[w, blk], dstv)

        def fire(kk, b):
            return [
                pltpu.async_copy(tsrc_hbm.at[srcv.at[kk]], gsrc[b],
                                 semg[b][0]),
                pltpu.async_copy(tdst_hbm.at[dstv.at[kk]], scb[b],
                                 semg[b][1]),
                pltpu.async_copy(eg_hbm.at[pl.ds(base + (k0 + kk) * CB, CB)],
                                 egb[b], semg[b][2]),
            ]

        def compute(b):
            gb, sb, eb = gsrc[b], scb[b], egb[b]

            def _one(i, st):
                nst = []
                for j in range(D // L):
                    sl = pl.ds(j * L, L)
                    sh = pl.ds(D + j * L, L)
                    m = gb[i, sl] + sb[i, sl] + eb[i, sl]
                    eb[i, sl] = m
                    sg = 1.0 / (1.0 + jnp.exp(-m))
                    sb[i, sl] = gb[i, sh] * sg
                    nst.append(st[j] + m)
                    nst.append(st[D // L + j] + m * m)
                return tuple(nst[::2]) + tuple(nst[1::2])

            def _row(p, st):
                st = _one(2 * p, st)
                st = _one(2 * p + 1, st)
                return st

            z = (zero16,) * (2 * (D // L))
            st = lax.fori_loop(0, CB // 2, _row, z)
            for j in range(D // L):
                sl = pl.ds(j * L, L)
                stats_v[0, sl] = stats_v[0, sl] + st[j]
                stats_v[1, sl] = stats_v[1, sl] + st[D // L + j]

        def fire_w(kk, b):
            return [
                pltpu.async_copy(egb[b],
                                 m_hbm.at[pl.ds(base + (k0 + kk) * CB, CB)],
                                 semw[b][0]),
                pltpu.async_copy(scb[b], shared_acc.at[dstv.at[kk]],
                                 semw[b][1], add=True),
            ]

        _pipeline(fire, compute, fire_w)

    def _phase2_block(blk):
        k0 = blk * CPB
        pltpu.sync_copy(dst_hbm.at[w, blk], dstv)

        def fire(kk, b):
            return [
                pltpu.async_copy(m_hbm.at[pl.ds(base + (k0 + kk) * CB, CB)],
                                 egb[b], semg[b][2]),
            ]

        def compute(b):
            sb, eb = scb[b], egb[b]

            def _one(i):
                for j in range(D // L):
                    sl = pl.ds(j * L, L)
                    sb[i, sl] = 1.0 / (1.0 + jnp.exp(-eb[i, sl]))

            def _row(p, rc):
                _one(2 * p)
                _one(2 * p + 1)
                return rc
            lax.fori_loop(0, CB // 2, _row, 0)

        def fire_w(kk, b):
            return [
                pltpu.async_copy(scb[b], shared_acc.at[dstv.at[kk]],
                                 semw[b][1], add=True),
            ]

        _pipeline(fire, compute, fire_w)

    _zero_shared()
    plsc.subcore_barrier()

    def _p1(blk, carry):
        _phase1_block(blk)
        return carry
    lax.fori_loop(0, IBLK, _p1, 0)
    plsc.subcore_barrier()
    pltpu.sync_copy(shared_acc.at[pl.ds(s * RPT, RPT)],
                    accm_hbm.at[c, pl.ds(s * RPT, RPT)])
    pltpu.sync_copy(stats_v, stats_hbm.at[c, s])
    _zero_shared()
    plsc.subcore_barrier()

    def _p2(blk, carry):
        _phase2_block(blk)
        return carry
    lax.fori_loop(0, IBLK, _p2, 0)
    plsc.subcore_barrier()
    pltpu.sync_copy(shared_acc.at[pl.ds(s * RPT, RPT)],
                    accs_hbm.at[c, pl.ds(s * RPT, RPT)])


def _sc_edge(src4, dst4, tsrc, tdst, eg):
    mesh = plsc.VectorSubcoreMesh(core_axis_name="c", subcore_axis_name="s",
                                  num_cores=NC, num_subcores=NS)
    fn = pl.kernel(
        _sc_edge_body,
        out_type=[
            jax.ShapeDtypeStruct((E, D), jnp.float32),
            jax.ShapeDtypeStruct((NC, N_PAD, D), jnp.float32),
            jax.ShapeDtypeStruct((NC, N_PAD, D), jnp.float32),
            jax.ShapeDtypeStruct((NC, NS, 2, D), jnp.float32),
        ],
        mesh=mesh,
        scratch_types=[
            pltpu.VMEM_SHARED((N_PAD, D), jnp.float32),
            pltpu.VMEM((CPB, CB), jnp.int32),
            pltpu.VMEM((CPB, CB), jnp.int32),
            pltpu.VMEM((2, CB, 2 * D), jnp.float32),
            pltpu.VMEM((2, CB, D), jnp.float32),
            pltpu.VMEM((2, CB, D), jnp.float32),
            pltpu.VMEM((2, D), jnp.float32),
            pltpu.SemaphoreType.DMA,
            pltpu.SemaphoreType.DMA,
            pltpu.SemaphoreType.DMA,
            pltpu.SemaphoreType.DMA,
            pltpu.SemaphoreType.DMA,
            pltpu.SemaphoreType.DMA,
            pltpu.SemaphoreType.DMA,
            pltpu.SemaphoreType.DMA,
            pltpu.SemaphoreType.DMA,
            pltpu.SemaphoreType.DMA,
        ],
    )
    return fn(src4, dst4, tsrc, tdst, eg)


def _node_final_body(ah_ref, accm_ref, accs_ref, nf_ref, g_ref, b_ref, x_ref):
    accm = accm_ref[...]
    accs = accs_ref[...]
    num = accm[0, :N, :] + accm[1, :N, :]
    den = accs[0, :N, :] + accs[1, :N, :]
    xp = ah_ref[...] + num / (den + 1e-6)
    mu = jnp.mean(xp, axis=0, keepdims=True)
    var = jnp.mean(xp * xp, axis=0, keepdims=True) - mu * mu
    xn = (xp - mu) * lax.rsqrt(var + 1e-5) * g_ref[...] + b_ref[...]
    x_ref[...] = nf_ref[...] + xn * jax.nn.sigmoid(xn)


def _node_final(ah, accm, accs, node_feats, gamma_n, beta_n):
    return pl.pallas_call(
        _node_final_body,
        out_shape=jax.ShapeDtypeStruct((N, D), jnp.float32),
    )(ah, accm, accs, node_feats, gamma_n.reshape(1, D), beta_n.reshape(1, D))


def _edge_final_body(m_ref, ef_ref, stats_ref, g_ref, b_ref, y_ref):
    st = stats_ref[...].reshape(NW, 2, D)
    mean = jnp.sum(st[:, 0, :], axis=0, keepdims=True) / E
    msq = jnp.sum(st[:, 1, :], axis=0, keepdims=True) / E
    var = msq - mean * mean
    m = m_ref[...]
    yn = (m - mean) * lax.rsqrt(var + 1e-5) * g_ref[...] + b_ref[...]
    y_ref[...] = ef_ref[...] + yn * jax.nn.sigmoid(yn)


def _edge_final(m, edge_feats, stats, gamma_e, beta_e):
    eblk = E // _EBLK
    return pl.pallas_call(
        _edge_final_body,
        grid=(eblk,),
        in_specs=[
            pl.BlockSpec((_EBLK, D), lambda i: (i, 0)),
            pl.BlockSpec((_EBLK, D), lambda i: (i, 0)),
            pl.BlockSpec((NC, NS, 2, D), lambda i: (0, 0, 0, 0)),
            pl.BlockSpec((1, D), lambda i: (0, 0)),
            pl.BlockSpec((1, D), lambda i: (0, 0)),
        ],
        out_specs=pl.BlockSpec((_EBLK, D), lambda i: (i, 0)),
        out_shape=jax.ShapeDtypeStruct((E, D), jnp.float32),
    )(m, edge_feats, stats, gamma_e.reshape(1, D), beta_e.reshape(1, D))


def kernel(node_feats, edge_feats, edge_index, Wsg, bsg, Wdg, bdg, Weg, beg,
           Wsu, bsu, Wdu, bdu, gamma_n, beta_n, gamma_e, beta_e):
    src = edge_index[0].astype(jnp.int32)
    dst = edge_index[1].astype(jnp.int32)

    tsrc, tdst, ah = _node_matmuls(
        node_feats, Wsg, bsg, Wdg, bdg, Wdu, bdu, Wsu, bsu)
    eg = _edge_gate(edge_feats, Weg, beg)

    m, accm, accs, stats = _sc_edge(src.reshape(NW, IBLK, CPB, CB),
                                    dst.reshape(NW, IBLK, CPB, CB),
                                    tsrc, tdst, eg)

    x = _node_final(ah, accm, accs, node_feats, gamma_n, beta_n)
    y = _edge_final(m, edge_feats, stats, gamma_e, beta_e)
    return (x, y)


# speedup vs baseline: 1.9087x; 1.4072x over previous
import functools

import jax
import jax.numpy as jnp
from jax import lax
from jax.experimental import pallas as pl
from jax.experimental.pallas import tpu as pltpu
from jax.experimental.pallas import tpu_sc as plsc

N = 10000
E = 320000
D = 128
NC = 2
NS = 16
L = 16
NW = NC * NS
EPW = E // NW
CB = 40
NCHUNK = EPW // CB
N_PAD = 10240
RPT = N_PAD // NS
IBLK = 25
CPB = NCHUNK // IBLK

_NBLK = 1000
_EBLK = 512


def _node_mm_body(x_ref, wsg_ref, bsg_ref, wdg_ref, bdg_ref, wdu_ref, bdu_ref,
                  wsu_ref, bsu_ref, tsrc_ref, tdst_ref, ah_ref):
    x = x_ref[...]
    rsg = jnp.dot(x, wsg_ref[...], preferred_element_type=jnp.float32) + bsg_ref[...]
    rdg = jnp.dot(x, wdg_ref[...], preferred_element_type=jnp.float32) + bdg_ref[...]
    rdu = jnp.dot(x, wdu_ref[...], preferred_element_type=jnp.float32) + bdu_ref[...]
    rsu = jnp.dot(x, wsu_ref[...], preferred_element_type=jnp.float32) + bsu_ref[...]
    tsrc_ref[:, :D] = rsg
    tsrc_ref[:, D:] = rdu
    tdst_ref[...] = rdg
    ah_ref[...] = rsu


def _node_matmuls(node_feats, Wsg, bsg, Wdg, bdg, Wdu, bdu, Wsu, bsu):
    nblk = N // _NBLK
    full_w = pl.BlockSpec((D, D), lambda i: (0, 0))
    full_b = pl.BlockSpec((1, D), lambda i: (0, 0))
    return pl.pallas_call(
        _node_mm_body,
        grid=(nblk,),
        in_specs=[
            pl.BlockSpec((_NBLK, D), lambda i: (i, 0)),
            full_w, full_b, full_w, full_b, full_w, full_b, full_w, full_b,
        ],
        out_specs=[pl.BlockSpec((_NBLK, 2 * D), lambda i: (i, 0)),
                   pl.BlockSpec((_NBLK, D), lambda i: (i, 0)),
                   pl.BlockSpec((_NBLK, D), lambda i: (i, 0))],
        out_shape=[
            jax.ShapeDtypeStruct((N, 2 * D), jnp.float32),
            jax.ShapeDtypeStruct((N, D), jnp.float32),
            jax.ShapeDtypeStruct((N, D), jnp.float32),
        ],
    )(node_feats, Wsg, bsg.reshape(1, D), Wdg, bdg.reshape(1, D),
      Wdu, bdu.reshape(1, D), Wsu, bsu.reshape(1, D))


def _edge_mm_body(x_ref, w_ref, b_ref, o_ref):
    o_ref[...] = (jnp.dot(x_ref[...], w_ref[...],
                          preferred_element_type=jnp.float32) + b_ref[...])


def _edge_gate(edge_feats, Weg, beg):
    eblk = E // _EBLK
    return pl.pallas_call(
        _edge_mm_body,
        grid=(eblk,),
        in_specs=[
            pl.BlockSpec((_EBLK, D), lambda i: (i, 0)),
            pl.BlockSpec((D, D), lambda i: (0, 0)),
            pl.BlockSpec((1, D), lambda i: (0, 0)),
        ],
        out_specs=pl.BlockSpec((_EBLK, D), lambda i: (i, 0)),
        out_shape=jax.ShapeDtypeStruct((E, D), jnp.float32),
    )(edge_feats, Weg, beg.reshape(1, D))


def _sc_edge_body(src_hbm, dst_hbm, tsrc_hbm, tdst_hbm, eg_hbm,
                  m_hbm, accm_hbm, accs_hbm,
                  shared_acc, srcv, dstv,
                  gsrc2, egb2, scb2,
                  sga0, sgb0, sge0, sga1, sgb1, sge1,
                  swm0, sws0, swm1, sws1):
    c = lax.axis_index("c")
    s = lax.axis_index("s")
    w = c * NS + s
    base = w * EPW
    zero16 = jnp.zeros((L,), jnp.float32)
    semg = ((sga0, sgb0, sge0), (sga1, sgb1, sge1))
    semw = ((swm0, sws0), (swm1, sws1))
    gsrc = (gsrc2.at[0], gsrc2.at[1])
    egb = (egb2.at[0], egb2.at[1])
    scb = (scb2.at[0], scb2.at[1])

    def _zrow(i, carry):
        for b in range(2):
            for j in range(D // L):
                scb2[b, i, pl.ds(j * L, L)] = zero16
        return carry

    def _zero_shared():
        lax.fori_loop(0, CB, _zrow, 0)
        for t in range(RPT // CB):
            pltpu.sync_copy(scb2.at[0],
                            shared_acc.at[pl.ds(s * RPT + t * CB, CB)])

    def _pipeline(fire, compute, fire_w):
        def _pair(p, carry):
            kk = 2 * p
            da = fire(kk, 0)
            for d in da:
                d.wait()
            db = fire(kk + 1, 1)
            compute(0)
            wa = fire_w(kk, 0)
            for d in db:
                d.wait()
            compute(1)
            for d in wa:
                d.wait()
            wb = fire_w(kk + 1, 1)
            for d in wb:
                d.wait()
            return carry
        lax.fori_loop(0, CPB // 2, _pair, 0)

    def _phase1_block(blk):
        k0 = blk * CPB
        pltpu.sync_copy(src_hbm.at[w, blk], srcv)
        pltpu.sync_copy(dst_hbm.at[w, blk], dstv)

        def fire(kk, b):
            return [
                pltpu.async_copy(tsrc_hbm.at[srcv.at[kk]], gsrc[b],
                                 semg[b][0]),
                pltpu.async_copy(tdst_hbm.at[dstv.at[kk]], scb[b],
                                 semg[b][1]),
                pltpu.async_copy(eg_hbm.at[pl.ds(base + (k0 + kk) * CB, CB)],
                                 egb[b], semg[b][2]),
            ]

        def compute(b):
            gb, sb, eb = gsrc[b], scb[b], egb[b]

            @plsc.parallel_loop(0, CB, 1, unroll=4)
            def _row(i):
                for j in range(D // L):
                    sl = pl.ds(j * L, L)
                    sh = pl.ds(D + j * L, L)
                    m = gb[i, sl] + sb[i, sl] + eb[i, sl]
                    eb[i, sl] = m
                    sg = 1.0 / (1.0 + jnp.exp(-m))
                    sb[i, sl] = gb[i, sh] * sg

        def fire_w(kk, b):
            return [
                pltpu.async_copy(egb[b],
                                 m_hbm.at[pl.ds(base + (k0 + kk) * CB, CB)],
                                 semw[b][0]),
                pltpu.async_copy(scb[b], shared_acc.at[dstv.at[kk]],
                                 semw[b][1], add=True),
            ]

        _pipeline(fire, compute, fire_w)

    def _phase2_block(blk):
        k0 = blk * CPB
        pltpu.sync_copy(dst_hbm.at[w, blk], dstv)

        def fire(kk, b):
            return [
                pltpu.async_copy(m_hbm.at[pl.ds(base + (k0 + kk) * CB, CB)],
                                 egb[b], semg[b][2]),
            ]

        def compute(b):
            sb, eb = scb[b], egb[b]

            @plsc.parallel_loop(0, CB, 1, unroll=4)
            def _row(i):
                for j in range(D // L):
                    sl = pl.ds(j * L, L)
                    sb[i, sl] = 1.0 / (1.0 + jnp.exp(-eb[i, sl]))

        def fire_w(kk, b):
            return [
                pltpu.async_copy(scb[b], shared_acc.at[dstv.at[kk]],
                                 semw[b][1], add=True),
            ]

        _pipeline(fire, compute, fire_w)

    _zero_shared()
    plsc.subcore_barrier()

    def _p1(blk, carry):
        _phase1_block(blk)
        return carry
    lax.fori_loop(0, IBLK, _p1, 0)
    plsc.subcore_barrier()
    pltpu.sync_copy(shared_acc.at[pl.ds(s * RPT, RPT)],
                    accm_hbm.at[c, pl.ds(s * RPT, RPT)])
    _zero_shared()
    plsc.subcore_barrier()

    def _p2(blk, carry):
        _phase2_block(blk)
        return carry
    lax.fori_loop(0, IBLK, _p2, 0)
    plsc.subcore_barrier()
    pltpu.sync_copy(shared_acc.at[pl.ds(s * RPT, RPT)],
                    accs_hbm.at[c, pl.ds(s * RPT, RPT)])


def _sc_edge(src4, dst4, tsrc, tdst, eg):
    mesh = plsc.VectorSubcoreMesh(core_axis_name="c", subcore_axis_name="s",
                                  num_cores=NC, num_subcores=NS)
    fn = pl.kernel(
        _sc_edge_body,
        out_type=[
            jax.ShapeDtypeStruct((E, D), jnp.float32),
            jax.ShapeDtypeStruct((NC, N_PAD, D), jnp.float32),
            jax.ShapeDtypeStruct((NC, N_PAD, D), jnp.float32),
        ],
        mesh=mesh,
        scratch_types=[
            pltpu.VMEM_SHARED((N_PAD, D), jnp.float32),
            pltpu.VMEM((CPB, CB), jnp.int32),
            pltpu.VMEM((CPB, CB), jnp.int32),
            pltpu.VMEM((2, CB, 2 * D), jnp.float32),
            pltpu.VMEM((2, CB, D), jnp.float32),
            pltpu.VMEM((2, CB, D), jnp.float32),
            pltpu.SemaphoreType.DMA,
            pltpu.SemaphoreType.DMA,
            pltpu.SemaphoreType.DMA,
            pltpu.SemaphoreType.DMA,
            pltpu.SemaphoreType.DMA,
            pltpu.SemaphoreType.DMA,
            pltpu.SemaphoreType.DMA,
            pltpu.SemaphoreType.DMA,
            pltpu.SemaphoreType.DMA,
            pltpu.SemaphoreType.DMA,
        ],
    )
    return fn(src4, dst4, tsrc, tdst, eg)


def _node_final_body(ah_ref, accm_ref, accs_ref, nf_ref, g_ref, b_ref, x_ref):
    accm = accm_ref[...]
    accs = accs_ref[...]
    num = accm[0, :N, :] + accm[1, :N, :]
    den = accs[0, :N, :] + accs[1, :N, :]
    xp = ah_ref[...] + num / (den + 1e-6)
    mu = jnp.mean(xp, axis=0, keepdims=True)
    var = jnp.mean(xp * xp, axis=0, keepdims=True) - mu * mu
    xn = (xp - mu) * lax.rsqrt(var + 1e-5) * g_ref[...] + b_ref[...]
    x_ref[...] = nf_ref[...] + xn * jax.nn.sigmoid(xn)


def _node_final(ah, accm, accs, node_feats, gamma_n, beta_n):
    return pl.pallas_call(
        _node_final_body,
        out_shape=jax.ShapeDtypeStruct((N, D), jnp.float32),
    )(ah, accm, accs, node_feats, gamma_n.reshape(1, D), beta_n.reshape(1, D))


def _edge_final_body(m_ref, ef_ref, g_ref, b_ref, y_ref, st_ref):
    p = pl.program_id(0)
    i = pl.program_id(1)
    m = m_ref[...]

    @pl.when(jnp.logical_and(p == 0, i == 0))
    def _init():
        st_ref[...] = jnp.zeros_like(st_ref)

    @pl.when(p == 0)
    def _accum():
        st_ref[0:1, :] += jnp.sum(m, axis=0, keepdims=True)
        st_ref[1:2, :] += jnp.sum(m * m, axis=0, keepdims=True)
        y_ref[...] = jnp.zeros_like(y_ref)

    @pl.when(p == 1)
    def _apply():
        mean = st_ref[0:1, :] / E
        var = st_ref[1:2, :] / E - mean * mean
        yn = (m - mean) * lax.rsqrt(var + 1e-5) * g_ref[...] + b_ref[...]
        y_ref[...] = ef_ref[...] + yn * jax.nn.sigmoid(yn)


def _edge_final(m, edge_feats, gamma_e, beta_e):
    eblk = E // _EBLK
    return pl.pallas_call(
        _edge_final_body,
        grid=(2, eblk),
        in_specs=[
            pl.BlockSpec((_EBLK, D), lambda p, i: (i, 0)),
            pl.BlockSpec((_EBLK, D), lambda p, i: (i * p, 0)),
            pl.BlockSpec((1, D), lambda p, i: (0, 0)),
            pl.BlockSpec((1, D), lambda p, i: (0, 0)),
        ],
        out_specs=pl.BlockSpec((_EBLK, D), lambda p, i: (i, 0)),
        out_shape=jax.ShapeDtypeStruct((E, D), jnp.float32),
        scratch_shapes=[pltpu.VMEM((2, D), jnp.float32)],
    )(m, edge_feats, gamma_e.reshape(1, D), beta_e.reshape(1, D))


def kernel(node_feats, edge_feats, edge_index, Wsg, bsg, Wdg, bdg, Weg, beg,
           Wsu, bsu, Wdu, bdu, gamma_n, beta_n, gamma_e, beta_e):
    src = edge_index[0].astype(jnp.int32)
    dst = edge_index[1].astype(jnp.int32)

    tsrc, tdst, ah = _node_matmuls(
        node_feats, Wsg, bsg, Wdg, bdg, Wdu, bdu, Wsu, bsu)
    eg = _edge_gate(edge_feats, Weg, beg)

    m, accm, accs = _sc_edge(src.reshape(NW, IBLK, CPB, CB),
                             dst.reshape(NW, IBLK, CPB, CB),
                             tsrc, tdst, eg)

    x = _node_final(ah, accm, accs, node_feats, gamma_n, beta_n)
    y = _edge_final(m, edge_feats, gamma_e, beta_e)
    return (x, y)


# speedup vs baseline: 2.7412x; 1.4362x over previous
import functools

import jax
import jax.numpy as jnp
from jax import lax
from jax.experimental import pallas as pl
from jax.experimental.pallas import tpu as pltpu
from jax.experimental.pallas import tpu_sc as plsc

N = 10000
E = 320000
D = 128
NC = 2
NS = 16
L = 16
NW = NC * NS
EPW = E // NW
CB = 40
NCHUNK = EPW // CB
N_PAD = 10240
RPT = N_PAD // NS
IBLK = 25
CPB = NCHUNK // IBLK

_NBLK = 2000
_EBLK = 2000


def _node_mm_body(x_ref, wsg_ref, bsg_ref, wdg_ref, bdg_ref, wdu_ref, bdu_ref,
                  wsu_ref, bsu_ref, tsrc_ref, tdst_ref, ah_ref):
    x = x_ref[...]
    rsg = jnp.dot(x, wsg_ref[...], preferred_element_type=jnp.float32) + bsg_ref[...]
    rdg = jnp.dot(x, wdg_ref[...], preferred_element_type=jnp.float32) + bdg_ref[...]
    rdu = jnp.dot(x, wdu_ref[...], preferred_element_type=jnp.float32) + bdu_ref[...]
    rsu = jnp.dot(x, wsu_ref[...], preferred_element_type=jnp.float32) + bsu_ref[...]
    tsrc_ref[:, :D] = rsg
    tsrc_ref[:, D:] = rdu
    tdst_ref[...] = rdg
    ah_ref[...] = rsu


def _node_matmuls(node_feats, Wsg, bsg, Wdg, bdg, Wdu, bdu, Wsu, bsu):
    nblk = N // _NBLK
    full_w = pl.BlockSpec((D, D), lambda i: (0, 0))
    full_b = pl.BlockSpec((1, D), lambda i: (0, 0))
    return pl.pallas_call(
        _node_mm_body,
        grid=(nblk,),
        in_specs=[
            pl.BlockSpec((_NBLK, D), lambda i: (i, 0)),
            full_w, full_b, full_w, full_b, full_w, full_b, full_w, full_b,
        ],
        out_specs=[pl.BlockSpec((_NBLK, 2 * D), lambda i: (i, 0)),
                   pl.BlockSpec((_NBLK, D), lambda i: (i, 0)),
                   pl.BlockSpec((_NBLK, D), lambda i: (i, 0))],
        out_shape=[
            jax.ShapeDtypeStruct((N, 2 * D), jnp.float32),
            jax.ShapeDtypeStruct((N, D), jnp.float32),
            jax.ShapeDtypeStruct((N, D), jnp.float32),
        ],
    )(node_feats, Wsg, bsg.reshape(1, D), Wdg, bdg.reshape(1, D),
      Wdu, bdu.reshape(1, D), Wsu, bsu.reshape(1, D))


def _edge_mm_body(x_ref, w_ref, b_ref, o_ref):
    o_ref[...] = (jnp.dot(x_ref[...], w_ref[...],
                          preferred_element_type=jnp.float32) + b_ref[...])


def _edge_gate(edge_feats, Weg, beg):
    eblk = E // _EBLK
    return pl.pallas_call(
        _edge_mm_body,
        grid=(eblk,),
        in_specs=[
            pl.BlockSpec((_EBLK, D), lambda i: (i, 0)),
            pl.BlockSpec((D, D), lambda i: (0, 0)),
            pl.BlockSpec((1, D), lambda i: (0, 0)),
        ],
        out_specs=pl.BlockSpec((_EBLK, D), lambda i: (i, 0)),
        out_shape=jax.ShapeDtypeStruct((E, D), jnp.float32),
    )(edge_feats, Weg, beg.reshape(1, D))


def _sc_edge_body(src_hbm, dst_hbm, tsrc_hbm, tdst_hbm, eg_hbm,
                  m_hbm, accm_hbm, accs_hbm,
                  shared_acc, srcv, dstv,
                  gsrc2, egb2, scb2,
                  sga0, sgb0, sge0, sga1, sgb1, sge1,
                  swm0, sws0, swm1, sws1):
    c = lax.axis_index("c")
    s = lax.axis_index("s")
    w = c * NS + s
    base = w * EPW
    zero16 = jnp.zeros((L,), jnp.float32)
    semg = ((sga0, sgb0, sge0), (sga1, sgb1, sge1))
    semw = ((swm0, sws0), (swm1, sws1))
    gsrc = (gsrc2.at[0], gsrc2.at[1])
    egb = (egb2.at[0], egb2.at[1])
    scb = (scb2.at[0], scb2.at[1])

    def _zrow(i, carry):
        for b in range(2):
            for j in range(D // L):
                scb2[b, i, pl.ds(j * L, L)] = zero16
        return carry

    def _zero_shared():
        lax.fori_loop(0, CB, _zrow, 0)
        for t in range(RPT // CB):
            pltpu.sync_copy(scb2.at[0],
                            shared_acc.at[pl.ds(s * RPT + t * CB, CB)])

    def _pipeline(fire, compute, fire_w):
        def _pair(p, carry):
            kk = 2 * p
            da = fire(kk, 0)
            for d in da:
                d.wait()
            db = fire(kk + 1, 1)
            compute(0)
            wa = fire_w(kk, 0)
            for d in db:
                d.wait()
            compute(1)
            for d in wa:
                d.wait()
            wb = fire_w(kk + 1, 1)
            for d in wb:
                d.wait()
            return carry
        lax.fori_loop(0, CPB // 2, _pair, 0)

    def _phase1_block(blk):
        k0 = blk * CPB
        pltpu.sync_copy(src_hbm.at[w, blk], srcv)
        pltpu.sync_copy(dst_hbm.at[w, blk], dstv)

        def fire(kk, b):
            return [
                pltpu.async_copy(tsrc_hbm.at[srcv.at[kk]], gsrc[b],
                                 semg[b][0]),
                pltpu.async_copy(tdst_hbm.at[dstv.at[kk]], scb[b],
                                 semg[b][1]),
                pltpu.async_copy(eg_hbm.at[pl.ds(base + (k0 + kk) * CB, CB)],
                                 egb[b], semg[b][2]),
            ]

        def compute(b):
            gb, sb, eb = gsrc[b], scb[b], egb[b]

            @plsc.parallel_loop(0, CB, 1, unroll=4)
            def _row(i):
                for j in range(D // L):
                    sl = pl.ds(j * L, L)
                    sh = pl.ds(D + j * L, L)
                    m = gb[i, sl] + sb[i, sl] + eb[i, sl]
                    eb[i, sl] = m
                    sg = 1.0 / (1.0 + jnp.exp(-m))
                    sb[i, sl] = gb[i, sh] * sg

        def fire_w(kk, b):
            return [
                pltpu.async_copy(egb[b],
                                 m_hbm.at[pl.ds(base + (k0 + kk) * CB, CB)],
                                 semw[b][0]),
                pltpu.async_copy(scb[b], shared_acc.at[dstv.at[kk]],
                                 semw[b][1], add=True),
            ]

        _pipeline(fire, compute, fire_w)

    def _phase2_block(blk):
        k0 = blk * CPB
        pltpu.sync_copy(dst_hbm.at[w, blk], dstv)

        def fire(kk, b):
            return [
                pltpu.async_copy(m_hbm.at[pl.ds(base + (k0 + kk) * CB, CB)],
                                 egb[b], semg[b][2]),
            ]

        def compute(b):
            sb, eb = scb[b], egb[b]

            @plsc.parallel_loop(0, CB, 1, unroll=4)
            def _row(i):
                for j in range(D // L):
                    sl = pl.ds(j * L, L)
                    sb[i, sl] = 1.0 / (1.0 + jnp.exp(-eb[i, sl]))

        def fire_w(kk, b):
            return [
                pltpu.async_copy(scb[b], shared_acc.at[dstv.at[kk]],
                                 semw[b][1], add=True),
            ]

        _pipeline(fire, compute, fire_w)

    _zero_shared()
    plsc.subcore_barrier()

    def _p1(blk, carry):
        _phase1_block(blk)
        return carry
    lax.fori_loop(0, IBLK, _p1, 0)
    plsc.subcore_barrier()
    pltpu.sync_copy(shared_acc.at[pl.ds(s * RPT, RPT)],
                    accm_hbm.at[c, pl.ds(s * RPT, RPT)])
    _zero_shared()
    plsc.subcore_barrier()

    def _p2(blk, carry):
        _phase2_block(blk)
        return carry
    lax.fori_loop(0, IBLK, _p2, 0)
    plsc.subcore_barrier()
    pltpu.sync_copy(shared_acc.at[pl.ds(s * RPT, RPT)],
                    accs_hbm.at[c, pl.ds(s * RPT, RPT)])


def _sc_edge(src4, dst4, tsrc, tdst, eg):
    mesh = plsc.VectorSubcoreMesh(core_axis_name="c", subcore_axis_name="s",
                                  num_cores=NC, num_subcores=NS)
    fn = pl.kernel(
        _sc_edge_body,
        out_type=[
            jax.ShapeDtypeStruct((E, D), jnp.float32),
            jax.ShapeDtypeStruct((NC, N_PAD, D), jnp.float32),
            jax.ShapeDtypeStruct((NC, N_PAD, D), jnp.float32),
        ],
        mesh=mesh,
        scratch_types=[
            pltpu.VMEM_SHARED((N_PAD, D), jnp.float32),
            pltpu.VMEM((CPB, CB), jnp.int32),
            pltpu.VMEM((CPB, CB), jnp.int32),
            pltpu.VMEM((2, CB, 2 * D), jnp.float32),
            pltpu.VMEM((2, CB, D), jnp.float32),
            pltpu.VMEM((2, CB, D), jnp.float32),
            pltpu.SemaphoreType.DMA,
            pltpu.SemaphoreType.DMA,
            pltpu.SemaphoreType.DMA,
            pltpu.SemaphoreType.DMA,
            pltpu.SemaphoreType.DMA,
            pltpu.SemaphoreType.DMA,
            pltpu.SemaphoreType.DMA,
            pltpu.SemaphoreType.DMA,
            pltpu.SemaphoreType.DMA,
            pltpu.SemaphoreType.DMA,
        ],
    )
    return fn(src4, dst4, tsrc, tdst, eg)


def _node_final_body(ah_ref, accm_ref, accs_ref, nf_ref, g_ref, b_ref, x_ref):
    accm = accm_ref[...]
    accs = accs_ref[...]
    num = accm[0, :N, :] + accm[1, :N, :]
    den = accs[0, :N, :] + accs[1, :N, :]
    xp = ah_ref[...] + num / (den + 1e-6)
    mu = jnp.mean(xp, axis=0, keepdims=True)
    var = jnp.mean(xp * xp, axis=0, keepdims=True) - mu * mu
    xn = (xp - mu) * lax.rsqrt(var + 1e-5) * g_ref[...] + b_ref[...]
    x_ref[...] = nf_ref[...] + xn * jax.nn.sigmoid(xn)


def _node_final(ah, accm, accs, node_feats, gamma_n, beta_n):
    return pl.pallas_call(
        _node_final_body,
        out_shape=jax.ShapeDtypeStruct((N, D), jnp.float32),
    )(ah, accm, accs, node_feats, gamma_n.reshape(1, D), beta_n.reshape(1, D))


def _edge_final_body(m_ref, ef_ref, g_ref, b_ref, y_ref, st_ref):
    p = pl.program_id(0)
    i = pl.program_id(1)
    m = m_ref[...]

    @pl.when(jnp.logical_and(p == 0, i == 0))
    def _init():
        st_ref[...] = jnp.zeros_like(st_ref)

    @pl.when(p == 0)
    def _accum():
        st_ref[0:1, :] += jnp.sum(m, axis=0, keepdims=True)
        st_ref[1:2, :] += jnp.sum(m * m, axis=0, keepdims=True)
        y_ref[...] = jnp.zeros_like(y_ref)

    @pl.when(p == 1)
    def _apply():
        mean = st_ref[0:1, :] / E
        var = st_ref[1:2, :] / E - mean * mean
        yn = (m - mean) * lax.rsqrt(var + 1e-5) * g_ref[...] + b_ref[...]
        y_ref[...] = ef_ref[...] + yn * jax.nn.sigmoid(yn)


def _edge_final(m, edge_feats, gamma_e, beta_e):
    eblk = E // _EBLK
    return pl.pallas_call(
        _edge_final_body,
        grid=(2, eblk),
        in_specs=[
            pl.BlockSpec((_EBLK, D), lambda p, i: (i, 0)),
            pl.BlockSpec((_EBLK, D), lambda p, i: (i * p, 0)),
            pl.BlockSpec((1, D), lambda p, i: (0, 0)),
            pl.BlockSpec((1, D), lambda p, i: (0, 0)),
        ],
        out_specs=pl.BlockSpec((_EBLK, D), lambda p, i: (i, 0)),
        out_shape=jax.ShapeDtypeStruct((E, D), jnp.float32),
        scratch_shapes=[pltpu.VMEM((2, D), jnp.float32)],
    )(m, edge_feats, gamma_e.reshape(1, D), beta_e.reshape(1, D))


def kernel(node_feats, edge_feats, edge_index, Wsg, bsg, Wdg, bdg, Weg, beg,
           Wsu, bsu, Wdu, bdu, gamma_n, beta_n, gamma_e, beta_e):
    src = edge_index[0].astype(jnp.int32)
    dst = edge_index[1].astype(jnp.int32)

    tsrc, tdst, ah = _node_matmuls(
        node_feats, Wsg, bsg, Wdg, bdg, Wdu, bdu, Wsu, bsu)
    eg = _edge_gate(edge_feats, Weg, beg)

    m, accm, accs = _sc_edge(src.reshape(NW, IBLK, CPB, CB),
                             dst.reshape(NW, IBLK, CPB, CB),
                             tsrc, tdst, eg)

    x = _node_final(ah, accm, accs, node_feats, gamma_n, beta_n)
    y = _edge_final(m, edge_feats, gamma_e, beta_e)
    return (x, y)
